# baseline (speedup 1.0000x reference)
import jax
import jax.numpy as jnp
from jax.experimental import pallas as pl
from jax.experimental.pallas import tpu as pltpu


def _loss_body(reg_ref, t_ref, p_ref, w_ref, e_ref, out_ref,
               acc0_ref, acc1_ref, acc2_ref):
    i = pl.program_id(0)
    n = pl.num_programs(0)

    @pl.when(i == 0)
    def _init():
        acc0_ref[...] = jnp.zeros_like(acc0_ref)
        acc1_ref[...] = jnp.zeros_like(acc1_ref)
        acc2_ref[...] = jnp.zeros_like(acc2_ref)

    d = t_ref[...] - p_ref[...]
    acc0_ref[...] += jnp.sum((d * d).reshape(-1, 8, 128), axis=0)
    w = w_ref[...]
    acc1_ref[...] += jnp.sum((w * jnp.maximum(w, 0.0)).reshape(-1, 8, 128),
                             axis=0)
    e = e_ref[...]
    acc2_ref[...] += jnp.sum((e * jnp.maximum(e, 0.0)).reshape(-1, 8, 128),
                             axis=0)

    @pl.when(i == n - 1)
    def _fin():
        out_ref[0, 0] = (0.5 * jnp.sqrt(jnp.sum(acc0_ref[...]))
                         + reg_ref[2] * (jnp.sqrt(jnp.sum(acc1_ref[...]))
                                         + jnp.sqrt(jnp.sum(acc2_ref[...]))))


def kernel(target, prediction, reg, batch, W, E, Sw, Se):
    N, D = target.shape
    Wt = W.T
    BLK = 2000
    grid = N // BLK

    rowblk = pl.BlockSpec((BLK, D), lambda i: (i, 0))
    out = pl.pallas_call(
        _loss_body,
        grid=(grid,),
        in_specs=[
            pl.BlockSpec(memory_space=pltpu.SMEM),
            rowblk, rowblk, rowblk, rowblk,
        ],
        out_specs=pl.BlockSpec(memory_space=pltpu.SMEM),
        out_shape=jax.ShapeDtypeStruct((1, 1), jnp.float32),
        scratch_shapes=[pltpu.VMEM((8, 128), jnp.float32)] * 3,
        compiler_params=pltpu.CompilerParams(
            dimension_semantics=("arbitrary",)),
    )(reg, target, prediction, Wt, E)
    return out[0, 0]


# speedup vs baseline: 1.1385x; 1.1385x over previous
import jax
import jax.numpy as jnp
from jax.experimental import pallas as pl
from jax.experimental.pallas import tpu as pltpu


def _loss_body(reg_ref, t_ref, p_ref, w_ref, e_ref, out_ref,
               acc0_ref, acc1_ref, acc2_ref):
    i = pl.program_id(0)
    n = pl.num_programs(0)

    @pl.when(i == 0)
    def _init():
        acc0_ref[...] = jnp.zeros_like(acc0_ref)
        acc1_ref[...] = jnp.zeros_like(acc1_ref)
        acc2_ref[...] = jnp.zeros_like(acc2_ref)

    d = t_ref[...] - p_ref[...]
    acc0_ref[...] += jnp.sum((d * d).reshape(-1, 8, 128), axis=0)
    w = w_ref[...]
    acc1_ref[...] += jnp.sum((w * jnp.maximum(w, 0.0)).reshape(-1, 8, 128),
                             axis=0)
    e = e_ref[...]
    acc2_ref[...] += jnp.sum((e * jnp.maximum(e, 0.0)).reshape(-1, 8, 128),
                             axis=0)

    @pl.when(i == n - 1)
    def _fin():
        out_ref[0, 0] = (0.5 * jnp.sqrt(jnp.sum(acc0_ref[...]))
                         + reg_ref[2] * (jnp.sqrt(jnp.sum(acc1_ref[...]))
                                         + jnp.sqrt(jnp.sum(acc2_ref[...]))))


def kernel(target, prediction, reg, batch, W, E, Sw, Se):
    N, D = target.shape
    Wt = W.T
    BLK = 4000
    grid = N // BLK

    rowblk = pl.BlockSpec((BLK, D), lambda i: (i, 0))
    out = pl.pallas_call(
        _loss_body,
        grid=(grid,),
        in_specs=[
            pl.BlockSpec(memory_space=pltpu.SMEM),
            rowblk, rowblk, rowblk, rowblk,
        ],
        out_specs=pl.BlockSpec(memory_space=pltpu.SMEM),
        out_shape=jax.ShapeDtypeStruct((1, 1), jnp.float32),
        scratch_shapes=[pltpu.VMEM((8, 128), jnp.float32)] * 3,
        compiler_params=pltpu.CompilerParams(
            dimension_semantics=("arbitrary",)),
    )(reg, target, prediction, Wt, E)
    return out[0, 0]
